# baseline (speedup 1.0000x reference)
import functools

import jax
import jax.numpy as jnp
from jax.experimental import pallas as pl

IN_DIM = 2048
OUT_DIM = 2048
R = 64
E = 8
TOP_K = 2
ALPHA = 16.0
SCALING = ALPHA / R


def _moe_half(xb, wr, wc, re2):
    tb = xb.shape[0]
    xb16 = xb.astype(jnp.bfloat16)
    ceT = jax.lax.dot_general(
        wc, xb16, (((1,), (1,)), ((), ())),
        preferred_element_type=jnp.float32,
    )
    logitsT = jax.lax.dot_general(
        wr, xb, (((1,), (1,)), ((), ())),
        preferred_element_type=jnp.float32,
    )

    ii = jax.lax.broadcasted_iota(jnp.int32, (E, tb), 0)
    m1 = jnp.max(logitsT, axis=0, keepdims=True)
    i1 = jnp.min(jnp.where(logitsT == m1, ii, E), axis=0, keepdims=True)
    masked = jnp.where(ii == i1, -jnp.inf, logitsT)
    m2 = jnp.max(masked, axis=0, keepdims=True)
    i2 = jnp.min(jnp.where(masked == m2, ii, E), axis=0, keepdims=True)
    w1 = SCALING / (1.0 + jnp.exp(m2 - m1))
    w2 = SCALING - w1
    wsel = jnp.where(ii == i1, w1, jnp.where(ii == i2, w2, 0.0))
    wsel16 = wsel.astype(jnp.bfloat16)
    ce16 = ceT.astype(jnp.bfloat16)

    zT = jnp.concatenate(
        [wsel16[e:e + 1, :] * ce16 for e in range(E)], axis=0)

    return jax.lax.dot_general(
        zT, re2, (((0,), (0,)), ((), ())),
        preferred_element_type=jnp.float32,
    )


def _moe_body(x_ref, wr_ref, wc_ref, re2_ref, o_ref, *, tb):
    tb2 = tb // 2
    wr, wc, re2 = wr_ref[...], wc_ref[...], re2_ref[...]
    o_ref[:tb2, :] = _moe_half(x_ref[:tb2, :], wr, wc, re2)
    o_ref[tb2:, :] = _moe_half(x_ref[tb2:, :], wr, wc, re2)


def _moe(xf, wr, wc, re2, *, tb, interpret=False):
    n_tok = xf.shape[0]
    return pl.pallas_call(
        functools.partial(_moe_body, tb=tb),
        grid=(n_tok // tb,),
        in_specs=[
            pl.BlockSpec((tb, IN_DIM), lambda i: (i, 0)),
            pl.BlockSpec((E, IN_DIM), lambda i: (0, 0)),
            pl.BlockSpec((R, IN_DIM), lambda i: (0, 0)),
            pl.BlockSpec((E * R, OUT_DIM), lambda i: (0, 0)),
        ],
        out_specs=pl.BlockSpec((tb, OUT_DIM), lambda i: (i, 0)),
        out_shape=jax.ShapeDtypeStruct((n_tok, OUT_DIM), jnp.float32),
        interpret=interpret,
    )(xf, wr, wc, re2)


def kernel(x, W_route, W_compress, routed_experts):
    B, T, D = x.shape
    xf = x.reshape(B * T, D)
    wc16 = W_compress.astype(jnp.bfloat16)
    re2 = (routed_experts.transpose(0, 2, 1)
           .reshape(E * R, OUT_DIM).astype(jnp.bfloat16))
    out = _moe(xf, W_route, wc16, re2, tb=1024)
    return out.reshape(B, T, OUT_DIM)


# speedup vs baseline: 1.0632x; 1.0632x over previous
import functools

import jax
import jax.numpy as jnp
from jax.experimental import pallas as pl

IN_DIM = 2048
OUT_DIM = 2048
R = 64
E = 8
TOP_K = 2
ALPHA = 16.0
SCALING = ALPHA / R


def _moe_half(xb, wrc, re2):
    tb = xb.shape[0]
    rT = jax.lax.dot_general(
        wrc, xb, (((1,), (1,)), ((), ())),
        preferred_element_type=jnp.float32,
    )
    ce16 = rT[:R, :].astype(jnp.bfloat16)
    logitsT = rT[R:, :]

    ii = jax.lax.broadcasted_iota(jnp.int32, (E, tb), 0)
    m1 = jnp.max(logitsT, axis=0, keepdims=True)
    i1 = jnp.min(jnp.where(logitsT == m1, ii, E), axis=0, keepdims=True)
    masked = jnp.where(ii == i1, -jnp.inf, logitsT)
    m2 = jnp.max(masked, axis=0, keepdims=True)
    i2 = jnp.min(jnp.where(masked == m2, ii, E), axis=0, keepdims=True)
    w1 = SCALING / (1.0 + jnp.exp(m2 - m1))
    w2 = SCALING - w1
    wsel16 = jnp.where(ii == i1, w1,
                       jnp.where(ii == i2, w2, 0.0)).astype(jnp.bfloat16)

    zT = jnp.concatenate(
        [wsel16[e:e + 1, :] * ce16 for e in range(E)], axis=0)

    return jax.lax.dot_general(
        zT, re2, (((0,), (0,)), ((), ())),
        preferred_element_type=jnp.float32,
    )


def _moe_body(x_ref, wrc_ref, re2_ref, o_ref, *, tb):
    tb2 = tb // 2
    wrc, re2 = wrc_ref[...], re2_ref[...]
    o_ref[:tb2, :] = _moe_half(x_ref[:tb2, :], wrc, re2)
    o_ref[tb2:, :] = _moe_half(x_ref[tb2:, :], wrc, re2)


def _moe(xf, wrc, re2, *, tb, interpret=False):
    n_tok = xf.shape[0]
    return pl.pallas_call(
        functools.partial(_moe_body, tb=tb),
        grid=(n_tok // tb,),
        in_specs=[
            pl.BlockSpec((tb, IN_DIM), lambda i: (i, 0)),
            pl.BlockSpec((R + E, IN_DIM), lambda i: (0, 0)),
            pl.BlockSpec((E * R, OUT_DIM), lambda i: (0, 0)),
        ],
        out_specs=pl.BlockSpec((tb, OUT_DIM), lambda i: (i, 0)),
        out_shape=jax.ShapeDtypeStruct((n_tok, OUT_DIM), jnp.float32),
        interpret=interpret,
    )(xf, wrc, re2)


def kernel(x, W_route, W_compress, routed_experts):
    B, T, D = x.shape
    xf = x.reshape(B * T, D)
    wrc = jnp.concatenate([W_compress, W_route], axis=0)
    re2 = (routed_experts.transpose(0, 2, 1)
           .reshape(E * R, OUT_DIM).astype(jnp.bfloat16))
    out = _moe(xf, wrc, re2, tb=1024)
    return out.reshape(B, T, OUT_DIM)
